# Initial kernel scaffold; baseline (speedup 1.0000x reference)
import jax
import jax.numpy as jnp
from jax.experimental import pallas as pl


def kernel(x, edge_index, W1, b1, W2, b2):
    raise NotImplementedError("write your pallas kernel here")


# speedup vs baseline: 13.4543x; 13.4543x over previous
import functools

import jax
import jax.numpy as jnp
from jax import lax
from jax.experimental import pallas as pl
from jax.experimental.pallas import tpu as pltpu
from jax.experimental.pallas import tpu_sc as plsc

N = 10000
E = 320000
NC = 2
NS = 16
NW = NC * NS
K = 40
NCH_T = E // NS // K
NCH_W = E // NW // K
N_PAD = 10240
RPT = N_PAD // NS
RBLK = 1000
GRID = N // RBLK

_mesh = plsc.VectorSubcoreMesh(core_axis_name="c", subcore_axis_name="s")


def _make_agg(CH):
    @functools.partial(
        pl.kernel,
        out_type=jax.ShapeDtypeStruct((NC, N_PAD, CH), jnp.float32),
        mesh=_mesh,
        compiler_params=pltpu.CompilerParams(use_tc_tiling_on_sc=False),
        scratch_types=[
            pltpu.VMEM((NCH_T, K), jnp.int32),
            pltpu.VMEM((NCH_T, K), jnp.int32),
            pltpu.VMEM((K, CH), jnp.float32),
            pltpu.VMEM((K, CH), jnp.float32),
            pltpu.VMEM_SHARED((N_PAD, CH), jnp.float32),
            pltpu.SemaphoreType.DMA,
            pltpu.SemaphoreType.DMA,
        ],
    )
    def agg(src_hbm, dst_hbm, table_hbm, zeros_hbm, out_hbm,
            src_v, dst_v, rows0, rows1, acc, g0, g1):
        cc = lax.axis_index("c")
        ss = lax.axis_index("s")

        pltpu.sync_copy(zeros_hbm, acc.at[pl.ds(ss * RPT, RPT)])
        plsc.subcore_barrier()

        pltpu.sync_copy(src_hbm.at[ss], src_v)
        pltpu.sync_copy(dst_hbm.at[ss], dst_v)

        tab = table_hbm.at[cc]

        pltpu.async_copy(tab.at[src_v.at[0]], rows0, g0)

        def pair(i, carry):
            j0 = 2 * i
            j1 = j0 + 1
            pltpu.make_async_copy(tab.at[src_v.at[j0]], rows0, g0).wait()
            pltpu.async_copy(tab.at[src_v.at[j1]], rows1, g1)
            pltpu.sync_copy(rows0, acc.at[dst_v.at[j0]], add=True)
            pltpu.make_async_copy(tab.at[src_v.at[j1]], rows1, g1).wait()
            j2 = lax.min(j1 + 1, NCH_T - 1)
            pltpu.async_copy(tab.at[src_v.at[j2]], rows0, g0)
            pltpu.sync_copy(rows1, acc.at[dst_v.at[j1]], add=True)
            return carry

        lax.fori_loop(0, NCH_T // 2, pair, 0)
        pltpu.make_async_copy(tab.at[src_v.at[0]], rows0, g0).wait()

        plsc.subcore_barrier()
        pltpu.sync_copy(acc.at[pl.ds(ss * RPT, RPT)],
                        out_hbm.at[cc, pl.ds(ss * RPT, RPT)])

    return agg


_agg64 = _make_agg(64)
_agg32 = _make_agg(32)

_DEGC = 16


@functools.partial(
    pl.kernel,
    out_type=jax.ShapeDtypeStruct((NC, N_PAD, _DEGC), jnp.float32),
    mesh=_mesh,
    compiler_params=pltpu.CompilerParams(use_tc_tiling_on_sc=False),
    scratch_types=[
        pltpu.VMEM((NCH_W, K), jnp.int32),
        pltpu.VMEM((K, _DEGC), jnp.float32),
        pltpu.VMEM_SHARED((N_PAD, _DEGC), jnp.float32),
    ],
)
def _deg(dst_hbm, ones_hbm, zeros_hbm, out_hbm, dst_v, ones_v, acc):
    cc = lax.axis_index("c")
    ss = lax.axis_index("s")
    wid = ss * NC + cc

    pltpu.sync_copy(zeros_hbm, acc.at[pl.ds(ss * RPT, RPT)])
    plsc.subcore_barrier()

    pltpu.sync_copy(dst_hbm.at[wid], dst_v)
    pltpu.sync_copy(ones_hbm, ones_v)

    def step(j, carry):
        pltpu.sync_copy(ones_v, acc.at[dst_v.at[j]], add=True)
        return carry

    lax.fori_loop(0, NCH_W, step, 0)

    plsc.subcore_barrier()
    pltpu.sync_copy(acc.at[pl.ds(ss * RPT, RPT)],
                    out_hbm.at[cc, pl.ds(ss * RPT, RPT)])


def _prep_body(degp_ref, x_ref, xp2_ref, dinv_ref):
    deg = degp_ref[0, :, 0:1] + degp_ref[1, :, 0:1] + 1.0
    dinv = lax.rsqrt(deg)
    xp = x_ref[...] * dinv
    xp2_ref[0] = xp[:, :64]
    xp2_ref[1] = xp[:, 64:]
    dinv_ref[...] = dinv


def _prep(degp, x):
    return pl.pallas_call(
        _prep_body,
        grid=(GRID,),
        in_specs=[
            pl.BlockSpec((NC, RBLK, _DEGC), lambda i: (0, i, 0)),
            pl.BlockSpec((RBLK, 128), lambda i: (i, 0)),
        ],
        out_specs=[
            pl.BlockSpec((NC, RBLK, 64), lambda i: (0, i, 0)),
            pl.BlockSpec((RBLK, 1), lambda i: (i, 0)),
        ],
        out_shape=[
            jax.ShapeDtypeStruct((NC, N, 64), jnp.float32),
            jax.ShapeDtypeStruct((N, 1), jnp.float32),
        ],
    )(degp, x)


def _dense_body(p_ref, xp2_ref, dinv_ref, W1_ref, b1_ref, W2_ref, out_ref):
    p = jnp.concatenate([p_ref[0], p_ref[1]], axis=1)
    xp = jnp.concatenate([xp2_ref[0], xp2_ref[1]], axis=1)
    agg = p + xp
    t = agg * dinv_ref[...]
    h1 = jnp.dot(t, W1_ref[...], preferred_element_type=jnp.float32) + b1_ref[...]
    h1 = jnp.maximum(h1, 0.0)
    h2 = jnp.dot(h1, W2_ref[...], preferred_element_type=jnp.float32)
    hp = h2 * dinv_ref[...]
    out_ref[0] = hp[:, :32]
    out_ref[1] = hp[:, 32:]


def _dense(p, xp2, dinv, W1, b1, W2):
    return pl.pallas_call(
        _dense_body,
        grid=(GRID,),
        in_specs=[
            pl.BlockSpec((NC, RBLK, 64), lambda i: (0, i, 0)),
            pl.BlockSpec((NC, RBLK, 64), lambda i: (0, i, 0)),
            pl.BlockSpec((RBLK, 1), lambda i: (i, 0)),
            pl.BlockSpec((128, 256), lambda i: (0, 0)),
            pl.BlockSpec((1, 256), lambda i: (0, 0)),
            pl.BlockSpec((256, 64), lambda i: (0, 0)),
        ],
        out_specs=pl.BlockSpec((NC, RBLK, 32), lambda i: (0, i, 0)),
        out_shape=jax.ShapeDtypeStruct((NC, N, 32), jnp.float32),
    )(p, xp2, dinv, W1, b1, W2)


def _final_body(q_ref, hp2_ref, dinv_ref, b2_ref, out_ref):
    q = jnp.concatenate([q_ref[0], q_ref[1]], axis=1)
    hp = jnp.concatenate([hp2_ref[0], hp2_ref[1]], axis=1)
    out_ref[...] = (q + hp) * dinv_ref[...] + b2_ref[...]


def _final(q, hp2, dinv, b2):
    return pl.pallas_call(
        _final_body,
        grid=(GRID,),
        in_specs=[
            pl.BlockSpec((NC, RBLK, 32), lambda i: (0, i, 0)),
            pl.BlockSpec((NC, RBLK, 32), lambda i: (0, i, 0)),
            pl.BlockSpec((RBLK, 1), lambda i: (i, 0)),
            pl.BlockSpec((1, 64), lambda i: (0, 0)),
        ],
        out_specs=pl.BlockSpec((RBLK, 64), lambda i: (i, 0)),
        out_shape=jax.ShapeDtypeStruct((N, 64), jnp.float32),
    )(q, hp2, dinv, b2)


@jax.jit
def _run(x, edge_index, W1, b1, W2, b2):
    src = edge_index[0].astype(jnp.int32)
    dst = edge_index[1].astype(jnp.int32)
    src_t = src.reshape(NS, NCH_T, K)
    dst_t = dst.reshape(NS, NCH_T, K)
    dst_w = dst.reshape(NW, NCH_W, K)

    degp = _deg(dst_w,
                jnp.ones((K, _DEGC), jnp.float32),
                jnp.zeros((RPT, _DEGC), jnp.float32))
    xp2, dinv = _prep(degp, x)
    p = _agg64(src_t, dst_t, xp2, jnp.zeros((RPT, 64), jnp.float32))
    hp2 = _dense(p, xp2, dinv, W1, b1.reshape(1, -1), W2)
    q = _agg32(src_t, dst_t, hp2, jnp.zeros((RPT, 32), jnp.float32))
    return _final(q, hp2, dinv, b2.reshape(1, -1))


def kernel(x, edge_index, W1, b1, W2, b2):
    return _run(x, edge_index, W1, b1, W2, b2)


# speedup vs baseline: 23.0651x; 1.7143x over previous
import functools

import jax
import jax.numpy as jnp
from jax import lax
from jax.experimental import pallas as pl
from jax.experimental.pallas import tpu as pltpu
from jax.experimental.pallas import tpu_sc as plsc

N = 10000
E = 320000
NC = 2
NS = 16
NW = NC * NS
K = 80
NCH_T = E // NS // K
NCH_W = E // NW // K
KD = 40
NCH_D = E // NW // KD
N_PAD = 10240
RPT = N_PAD // NS
RBLK = 1000
GRID = N // RBLK

_mesh = plsc.VectorSubcoreMesh(core_axis_name="c", subcore_axis_name="s")
_params = pltpu.CompilerParams(use_tc_tiling_on_sc=False)


def _make_agg(CH, NCH, col_split):
    idx_shape = (NS if col_split else NW, NCH, K)

    @functools.partial(
        pl.kernel,
        out_type=jax.ShapeDtypeStruct((NC, N_PAD, CH), jnp.float32),
        mesh=_mesh,
        compiler_params=_params,
        scratch_types=[
            pltpu.VMEM((NCH, K), jnp.int32),
            pltpu.VMEM((NCH, K), jnp.int32),
            pltpu.VMEM((K, CH), jnp.float32),
            pltpu.VMEM((K, CH), jnp.float32),
            pltpu.VMEM_SHARED((N_PAD, CH), jnp.float32),
            pltpu.SemaphoreType.DMA,
            pltpu.SemaphoreType.DMA,
        ],
    )
    def agg(src_hbm, dst_hbm, table_hbm, zeros_hbm, out_hbm,
            src_v, dst_v, rows0, rows1, acc, g0, g1):
        cc = lax.axis_index("c")
        ss = lax.axis_index("s")
        wid = ss if col_split else ss * NC + cc

        pltpu.sync_copy(zeros_hbm, acc.at[pl.ds(ss * RPT, RPT)])
        plsc.subcore_barrier()

        pltpu.sync_copy(src_hbm.at[wid], src_v)
        pltpu.sync_copy(dst_hbm.at[wid], dst_v)

        tab = table_hbm.at[cc] if col_split else table_hbm

        pltpu.async_copy(tab.at[src_v.at[0]], rows0, g0)

        def pair(i, carry):
            j0 = 2 * i
            j1 = j0 + 1
            pltpu.make_async_copy(tab.at[src_v.at[j0]], rows0, g0).wait()
            pltpu.async_copy(tab.at[src_v.at[j1]], rows1, g1)
            pltpu.sync_copy(rows0, acc.at[dst_v.at[j0]], add=True)
            pltpu.make_async_copy(tab.at[src_v.at[j1]], rows1, g1).wait()
            j2 = lax.min(j1 + 1, NCH - 1)
            pltpu.async_copy(tab.at[src_v.at[j2]], rows0, g0)
            pltpu.sync_copy(rows1, acc.at[dst_v.at[j1]], add=True)
            return carry

        lax.fori_loop(0, NCH // 2, pair, 0)
        pltpu.make_async_copy(tab.at[src_v.at[0]], rows0, g0).wait()
        if NCH % 2:
            pltpu.sync_copy(rows0, acc.at[dst_v.at[NCH - 1]], add=True)

        plsc.subcore_barrier()
        pltpu.sync_copy(acc.at[pl.ds(ss * RPT, RPT)],
                        out_hbm.at[cc, pl.ds(ss * RPT, RPT)])

    return agg


_agg1 = _make_agg(64, NCH_T, col_split=True)
_agg2 = _make_agg(64, NCH_W, col_split=False)

_DEGC = 16


@functools.partial(
    pl.kernel,
    out_type=jax.ShapeDtypeStruct((NC, N_PAD, _DEGC), jnp.float32),
    mesh=_mesh,
    compiler_params=_params,
    scratch_types=[
        pltpu.VMEM((NCH_D, KD), jnp.int32),
        pltpu.VMEM((KD, _DEGC), jnp.float32),
        pltpu.VMEM_SHARED((N_PAD, _DEGC), jnp.float32),
    ],
)
def _deg(dst_hbm, ones_hbm, zeros_hbm, out_hbm, dst_v, ones_v, acc):
    cc = lax.axis_index("c")
    ss = lax.axis_index("s")
    wid = ss * NC + cc

    pltpu.sync_copy(zeros_hbm, acc.at[pl.ds(ss * RPT, RPT)])
    plsc.subcore_barrier()

    pltpu.sync_copy(dst_hbm.at[wid], dst_v)
    pltpu.sync_copy(ones_hbm, ones_v)

    def step(j, carry):
        pltpu.sync_copy(ones_v, acc.at[dst_v.at[j]], add=True)
        return carry

    lax.fori_loop(0, NCH_D, step, 0)

    plsc.subcore_barrier()
    pltpu.sync_copy(acc.at[pl.ds(ss * RPT, RPT)],
                    out_hbm.at[cc, pl.ds(ss * RPT, RPT)])


def _prep_body(degp_ref, x_ref, xp2_ref, dinv_ref):
    deg = degp_ref[0, :, 0:1] + degp_ref[1, :, 0:1] + 1.0
    dinv = lax.rsqrt(deg)
    xp = x_ref[...] * dinv
    xp2_ref[0] = xp[:, :64]
    xp2_ref[1] = xp[:, 64:]
    dinv_ref[...] = dinv


def _prep(degp, x):
    return pl.pallas_call(
        _prep_body,
        grid=(GRID,),
        in_specs=[
            pl.BlockSpec((NC, RBLK, _DEGC), lambda i: (0, i, 0)),
            pl.BlockSpec((RBLK, 128), lambda i: (i, 0)),
        ],
        out_specs=[
            pl.BlockSpec((NC, RBLK, 64), lambda i: (0, i, 0)),
            pl.BlockSpec((RBLK, 1), lambda i: (i, 0)),
        ],
        out_shape=[
            jax.ShapeDtypeStruct((NC, N, 64), jnp.float32),
            jax.ShapeDtypeStruct((N, 1), jnp.float32),
        ],
    )(degp, x)


def _dense_body(p_ref, xp2_ref, dinv_ref, W1_ref, b1_ref, W2_ref, out_ref):
    p = jnp.concatenate([p_ref[0], p_ref[1]], axis=1)
    xp = jnp.concatenate([xp2_ref[0], xp2_ref[1]], axis=1)
    agg = p + xp
    t = agg * dinv_ref[...]
    h1 = jnp.dot(t, W1_ref[...], preferred_element_type=jnp.float32) + b1_ref[...]
    h1 = jnp.maximum(h1, 0.0)
    h2 = jnp.dot(h1, W2_ref[...], preferred_element_type=jnp.float32)
    out_ref[...] = h2 * dinv_ref[...]


def _dense(p, xp2, dinv, W1, b1, W2):
    return pl.pallas_call(
        _dense_body,
        grid=(GRID,),
        in_specs=[
            pl.BlockSpec((NC, RBLK, 64), lambda i: (0, i, 0)),
            pl.BlockSpec((NC, RBLK, 64), lambda i: (0, i, 0)),
            pl.BlockSpec((RBLK, 1), lambda i: (i, 0)),
            pl.BlockSpec((128, 256), lambda i: (0, 0)),
            pl.BlockSpec((1, 256), lambda i: (0, 0)),
            pl.BlockSpec((256, 64), lambda i: (0, 0)),
        ],
        out_specs=pl.BlockSpec((RBLK, 64), lambda i: (i, 0)),
        out_shape=jax.ShapeDtypeStruct((N, 64), jnp.float32),
    )(p, xp2, dinv, W1, b1, W2)


def _final_body(q_ref, hp_ref, dinv_ref, b2_ref, out_ref):
    out_ref[...] = ((q_ref[0] + q_ref[1] + hp_ref[...]) * dinv_ref[...]
                    + b2_ref[...])


def _final(q, hp, dinv, b2):
    return pl.pallas_call(
        _final_body,
        grid=(GRID,),
        in_specs=[
            pl.BlockSpec((NC, RBLK, 64), lambda i: (0, i, 0)),
            pl.BlockSpec((RBLK, 64), lambda i: (i, 0)),
            pl.BlockSpec((RBLK, 1), lambda i: (i, 0)),
            pl.BlockSpec((1, 64), lambda i: (0, 0)),
        ],
        out_specs=pl.BlockSpec((RBLK, 64), lambda i: (i, 0)),
        out_shape=jax.ShapeDtypeStruct((N, 64), jnp.float32),
    )(q, hp, dinv, b2)


@jax.jit
def _run(x, edge_index, W1, b1, W2, b2):
    src = edge_index[0].astype(jnp.int32)
    dst = edge_index[1].astype(jnp.int32)
    src_t = src.reshape(NS, NCH_T, K)
    dst_t = dst.reshape(NS, NCH_T, K)
    src_w = src.reshape(NW, NCH_W, K)
    dst_w = dst.reshape(NW, NCH_W, K)
    dst_d = dst.reshape(NW, NCH_D, KD)

    degp = _deg(dst_d,
                jnp.ones((KD, _DEGC), jnp.float32),
                jnp.zeros((RPT, _DEGC), jnp.float32))
    xp2, dinv = _prep(degp, x)
    p = _agg1(src_t, dst_t, xp2, jnp.zeros((RPT, 64), jnp.float32))
    hp = _dense(p, xp2, dinv, W1, b1.reshape(1, -1), W2)
    q = _agg2(src_w, dst_w, hp, jnp.zeros((RPT, 64), jnp.float32))
    return _final(q, hp, dinv, b2.reshape(1, -1))


def kernel(x, edge_index, W1, b1, W2, b2):
    return _run(x, edge_index, W1, b1, W2, b2)
